# Initial kernel scaffold; baseline (speedup 1.0000x reference)
import functools

import jax
import jax.numpy as jnp
from jax import lax
from jax.experimental import pallas as pl
from jax.experimental.pallas import tpu as pltpu
from jax.experimental.pallas import tpu_sc as plsc

NC = 2
NS = 16
NW = NC * NS


def _sc_mesh():
    return plsc.VectorSubcoreMesh(
        core_axis_name="c", subcore_axis_name="s",
        num_cores=NC, num_subcores=NS)


def _make_hist(E, MDEG, K):
    EPW = E // NW
    nch = EPW // K
    RPT = MDEG // NS

    @functools.partial(
        pl.kernel,
        out_type=jax.ShapeDtypeStruct((NC * MDEG,), jnp.float32),
        mesh=_sc_mesh(),
        scratch_types=[
            pltpu.VMEM_SHARED((MDEG,), jnp.float32),
            pltpu.VMEM((EPW,), jnp.int32),
            pltpu.VMEM((K,), jnp.float32),
            pltpu.VMEM((K,), jnp.int32),
            pltpu.VMEM((RPT,), jnp.float32),
        ],
    )
    def hist(dst_hbm, out_hbm, deg_sh, dste, ones_v, didx, zbuf):
        c = lax.axis_index("c")
        s = lax.axis_index("s")
        w = c * NS + s
        for i in range(RPT // 16):
            zbuf[pl.ds(i * 16, 16)] = jnp.zeros((16,), jnp.float32)
        for i in range(K // 16):
            ones_v[pl.ds(i * 16, 16)] = jnp.ones((16,), jnp.float32)
        pltpu.sync_copy(zbuf, deg_sh.at[pl.ds(s * RPT, RPT)])
        pltpu.sync_copy(dst_hbm.at[pl.ds(w * EPW, EPW)], dste)
        plsc.subcore_barrier()

        def body(j, carry):
            pltpu.sync_copy(dste.at[pl.ds(j * K, K)], didx)
            pltpu.sync_copy(ones_v, deg_sh.at[didx], add=True)
            return carry

        lax.fori_loop(0, nch, body, 0)
        plsc.subcore_barrier()
        pltpu.sync_copy(deg_sh.at[pl.ds(s * RPT, RPT)],
                        out_hbm.at[pl.ds(c * MDEG + s * RPT, RPT)])

    return hist


def _make_scatter(NPAD, D, E, K):
    EPW = E // NW
    nch = EPW // K
    RPT = NPAD // NS

    @functools.partial(
        pl.kernel,
        out_type=jax.ShapeDtypeStruct((NC * NPAD, D), jnp.float32),
        mesh=_sc_mesh(),
        scratch_types=[
            pltpu.VMEM_SHARED((NPAD, D), jnp.float32),
            pltpu.VMEM((EPW,), jnp.int32),
            pltpu.VMEM((EPW,), jnp.int32),
            pltpu.VMEM((K,), jnp.int32),
            pltpu.VMEM((K,), jnp.int32),
            pltpu.VMEM((K, D), jnp.float32),
            pltpu.SemaphoreType.DMA,
        ],
    )
    def scat(g_hbm, src_hbm, dst_hbm, zeros_hbm, out_hbm,
             acc_sh, srcb, dstb, sidx, didx, rows, sem):
        c = lax.axis_index("c")
        s = lax.axis_index("s")
        w = c * NS + s
        pltpu.sync_copy(zeros_hbm.at[pl.ds(s * RPT, RPT)],
                        acc_sh.at[pl.ds(s * RPT, RPT)])
        pltpu.sync_copy(src_hbm.at[pl.ds(w * EPW, EPW)], srcb)
        pltpu.sync_copy(dst_hbm.at[pl.ds(w * EPW, EPW)], dstb)
        plsc.subcore_barrier()

        def body(j, carry):
            pltpu.sync_copy(srcb.at[pl.ds(j * K, K)], sidx)
            pltpu.async_copy(g_hbm.at[sidx], rows, sem).wait()
            pltpu.sync_copy(dstb.at[pl.ds(j * K, K)], didx)
            pltpu.sync_copy(rows, acc_sh.at[didx], add=True)
            return carry

        lax.fori_loop(0, nch, body, 0)
        plsc.subcore_barrier()
        pltpu.sync_copy(acc_sh.at[pl.ds(s * RPT, RPT)],
                        out_hbm.at[pl.ds(c * NPAD + s * RPT, RPT)])

    return scat


def _matmul_scale(x, W, d0, d1):
    N, Din = x.shape
    Dout = W.shape[1]
    BN = 1000

    def body(x_ref, w_ref, d0_ref, d1_ref, g_ref, dinv_ref):
        dinv = lax.rsqrt(d0_ref[...] + d1_ref[...] + 1.0)
        h = jnp.dot(x_ref[...], w_ref[...],
                    preferred_element_type=jnp.float32)
        g_ref[...] = h * dinv
        dinv_ref[...] = dinv

    return pl.pallas_call(
        body,
        grid=(N // BN,),
        in_specs=[
            pl.BlockSpec((BN, Din), lambda i: (i, 0)),
            pl.BlockSpec((Din, Dout), lambda i: (0, 0)),
            pl.BlockSpec((BN, 1), lambda i: (i, 0)),
            pl.BlockSpec((BN, 1), lambda i: (i, 0)),
        ],
        out_specs=[
            pl.BlockSpec((BN, Dout), lambda i: (i, 0)),
            pl.BlockSpec((BN, 1), lambda i: (i, 0)),
        ],
        out_shape=[
            jax.ShapeDtypeStruct((N, Dout), jnp.float32),
            jax.ShapeDtypeStruct((N, 1), jnp.float32),
        ],
    )(x, W, d0, d1)


def _final(acc, g, dinv, b2d):
    N = g.shape[0]
    D = g.shape[1]
    BN = 1000

    def body(a_ref, g_ref, dinv_ref, b_ref, o_ref):
        o_ref[...] = (dinv_ref[...] * (a_ref[0] + a_ref[1] + g_ref[...])
                      + b_ref[...])

    return pl.pallas_call(
        body,
        grid=(N // BN,),
        in_specs=[
            pl.BlockSpec((2, BN, D), lambda i: (0, i, 0)),
            pl.BlockSpec((BN, D), lambda i: (i, 0)),
            pl.BlockSpec((BN, 1), lambda i: (i, 0)),
            pl.BlockSpec((1, D), lambda i: (0, 0)),
        ],
        out_specs=pl.BlockSpec((BN, D), lambda i: (i, 0)),
        out_shape=jax.ShapeDtypeStruct((N, D), jnp.float32),
    )(acc, g, dinv, b2d)


def kernel(x, edge_index, t_embed, W, b):
    N, Din = x.shape
    Dout = W.shape[1]
    E = edge_index.shape[1]
    src = edge_index[0]
    dst = edge_index[1]

    NPAD = 10240
    K = 80

    degp = _make_hist(E, NPAD, K)(dst)
    d0 = degp[:N].reshape(N, 1)
    d1 = degp[NPAD:NPAD + N].reshape(N, 1)

    g, dinv = _matmul_scale(x, W, d0, d1)

    zeros2d = jnp.zeros((NPAD, Dout), jnp.float32)
    acc = _make_scatter(NPAD, Dout, E, K)(g, src, dst, zeros2d)
    acc = acc.reshape(NC, NPAD, Dout)

    out = _final(acc, g, dinv, b.reshape(1, Dout))
    return (out, edge_index, t_embed)


# speedup vs baseline: 25.9573x; 25.9573x over previous
import functools

import jax
import jax.numpy as jnp
from jax import lax
from jax.experimental import pallas as pl
from jax.experimental.pallas import tpu as pltpu
from jax.experimental.pallas import tpu_sc as plsc

NC = 2
NS = 16
NW = NC * NS


def _sc_mesh():
    return plsc.VectorSubcoreMesh(
        core_axis_name="c", subcore_axis_name="s",
        num_cores=NC, num_subcores=NS)


def _make_hist(E, MDEG, K):
    EPW = E // NW
    nch = EPW // K
    RPT = MDEG // NS

    @functools.partial(
        pl.kernel,
        out_type=jax.ShapeDtypeStruct((NC * MDEG,), jnp.float32),
        mesh=_sc_mesh(),
        scratch_types=[
            pltpu.VMEM_SHARED((MDEG,), jnp.float32),
            pltpu.VMEM((nch, K), jnp.int32),
            pltpu.VMEM((K,), jnp.float32),
            pltpu.VMEM((RPT,), jnp.float32),
        ],
    )
    def hist(dst_hbm, out_hbm, deg_sh, dste, ones_v, zbuf):
        c = lax.axis_index("c")
        s = lax.axis_index("s")
        w = c * NS + s
        for i in range(RPT // 16):
            zbuf[pl.ds(i * 16, 16)] = jnp.zeros((16,), jnp.float32)
        for i in range(K // 16):
            ones_v[pl.ds(i * 16, 16)] = jnp.ones((16,), jnp.float32)
        pltpu.sync_copy(zbuf, deg_sh.at[pl.ds(s * RPT, RPT)])
        pltpu.sync_copy(dst_hbm.at[w], dste)
        plsc.subcore_barrier()

        def body(j, carry):
            pltpu.sync_copy(ones_v, deg_sh.at[dste.at[j]], add=True)
            return carry

        lax.fori_loop(0, nch, body, 0)
        plsc.subcore_barrier()
        pltpu.sync_copy(deg_sh.at[pl.ds(s * RPT, RPT)],
                        out_hbm.at[pl.ds(c * MDEG + s * RPT, RPT)])

    return hist


def _make_scatter(NPAD, D, E, K):
    EPW = E // NW
    nch = EPW // K
    RPT = NPAD // NS

    @functools.partial(
        pl.kernel,
        out_type=jax.ShapeDtypeStruct((NC * NPAD, D), jnp.float32),
        mesh=_sc_mesh(),
        scratch_types=[
            pltpu.VMEM_SHARED((NPAD, D), jnp.float32),
            pltpu.VMEM((nch, K), jnp.int32),
            pltpu.VMEM((nch, K), jnp.int32),
            pltpu.VMEM((K, D), jnp.float32),
            pltpu.SemaphoreType.DMA,
        ],
    )
    def scat(g_hbm, src_hbm, dst_hbm, zeros_hbm, out_hbm,
             acc_sh, srcb, dstb, rows, sem):
        c = lax.axis_index("c")
        s = lax.axis_index("s")
        w = c * NS + s
        pltpu.sync_copy(zeros_hbm.at[pl.ds(s * RPT, RPT)],
                        acc_sh.at[pl.ds(s * RPT, RPT)])
        pltpu.sync_copy(src_hbm.at[w], srcb)
        pltpu.sync_copy(dst_hbm.at[w], dstb)
        plsc.subcore_barrier()

        def body(j, carry):
            pltpu.async_copy(g_hbm.at[srcb.at[j]], rows, sem).wait()
            pltpu.sync_copy(rows, acc_sh.at[dstb.at[j]], add=True)
            return carry

        lax.fori_loop(0, nch, body, 0)
        plsc.subcore_barrier()
        pltpu.sync_copy(acc_sh.at[pl.ds(s * RPT, RPT)],
                        out_hbm.at[pl.ds(c * NPAD + s * RPT, RPT)])

    return scat


def _matmul_scale(x, W, d0, d1):
    N, Din = x.shape
    Dout = W.shape[1]
    BN = 1000

    def body(x_ref, w_ref, d0_ref, d1_ref, g_ref, dinv_ref):
        dinv = lax.rsqrt(d0_ref[...] + d1_ref[...] + 1.0)
        h = jnp.dot(x_ref[...], w_ref[...],
                    preferred_element_type=jnp.float32)
        g_ref[...] = h * dinv
        dinv_ref[...] = dinv

    return pl.pallas_call(
        body,
        grid=(N // BN,),
        in_specs=[
            pl.BlockSpec((BN, Din), lambda i: (i, 0)),
            pl.BlockSpec((Din, Dout), lambda i: (0, 0)),
            pl.BlockSpec((BN, 1), lambda i: (i, 0)),
            pl.BlockSpec((BN, 1), lambda i: (i, 0)),
        ],
        out_specs=[
            pl.BlockSpec((BN, Dout), lambda i: (i, 0)),
            pl.BlockSpec((BN, 1), lambda i: (i, 0)),
        ],
        out_shape=[
            jax.ShapeDtypeStruct((N, Dout), jnp.float32),
            jax.ShapeDtypeStruct((N, 1), jnp.float32),
        ],
    )(x, W, d0, d1)


def _final(acc, g, dinv, b2d):
    N = g.shape[0]
    D = g.shape[1]
    BN = 1000

    def body(a_ref, g_ref, dinv_ref, b_ref, o_ref):
        o_ref[...] = (dinv_ref[...] * (a_ref[0] + a_ref[1] + g_ref[...])
                      + b_ref[...])

    return pl.pallas_call(
        body,
        grid=(N // BN,),
        in_specs=[
            pl.BlockSpec((2, BN, D), lambda i: (0, i, 0)),
            pl.BlockSpec((BN, D), lambda i: (i, 0)),
            pl.BlockSpec((BN, 1), lambda i: (i, 0)),
            pl.BlockSpec((1, D), lambda i: (0, 0)),
        ],
        out_specs=pl.BlockSpec((BN, D), lambda i: (i, 0)),
        out_shape=jax.ShapeDtypeStruct((N, D), jnp.float32),
    )(acc, g, dinv, b2d)


def kernel(x, edge_index, t_embed, W, b):
    N, Din = x.shape
    Dout = W.shape[1]
    E = edge_index.shape[1]
    src = edge_index[0]
    dst = edge_index[1]

    NPAD = 10240
    K = 80

    nch = E // NW // K
    src3d = src.reshape(NW, nch, K)
    dst3d = dst.reshape(NW, nch, K)

    degp = _make_hist(E, NPAD, K)(dst3d)
    d0 = degp[:N].reshape(N, 1)
    d1 = degp[NPAD:NPAD + N].reshape(N, 1)

    g, dinv = _matmul_scale(x, W, d0, d1)

    zeros2d = jnp.zeros((NPAD, Dout), jnp.float32)
    acc = _make_scatter(NPAD, Dout, E, K)(g, src3d, dst3d, zeros2d)
    acc = acc.reshape(NC, NPAD, Dout)

    out = _final(acc, g, dinv, b.reshape(1, Dout))
    return (out, edge_index, t_embed)


# speedup vs baseline: 35.2240x; 1.3570x over previous
import functools

import jax
import jax.numpy as jnp
from jax import lax
from jax.experimental import pallas as pl
from jax.experimental.pallas import tpu as pltpu
from jax.experimental.pallas import tpu_sc as plsc

NC = 2
NS = 16
NW = NC * NS


def _sc_mesh():
    return plsc.VectorSubcoreMesh(
        core_axis_name="c", subcore_axis_name="s",
        num_cores=NC, num_subcores=NS)


def _make_hist(E, MDEG, K):
    EPW = E // NW
    nch = EPW // K
    RPT = MDEG // NS

    @functools.partial(
        pl.kernel,
        out_type=jax.ShapeDtypeStruct((NC * MDEG,), jnp.float32),
        mesh=_sc_mesh(),
        scratch_types=[
            pltpu.VMEM_SHARED((MDEG,), jnp.float32),
            pltpu.VMEM((nch, K), jnp.int32),
            pltpu.VMEM((K,), jnp.float32),
            pltpu.VMEM((RPT,), jnp.float32),
        ],
    )
    def hist(dst_hbm, out_hbm, deg_sh, dste, ones_v, zbuf):
        c = lax.axis_index("c")
        s = lax.axis_index("s")
        w = c * NS + s
        for i in range(RPT // 16):
            zbuf[pl.ds(i * 16, 16)] = jnp.zeros((16,), jnp.float32)
        for i in range(K // 16):
            ones_v[pl.ds(i * 16, 16)] = jnp.ones((16,), jnp.float32)
        pltpu.sync_copy(zbuf, deg_sh.at[pl.ds(s * RPT, RPT)])
        pltpu.sync_copy(dst_hbm.at[w], dste)
        plsc.subcore_barrier()

        def body(j, carry):
            pltpu.sync_copy(ones_v, deg_sh.at[dste.at[j]], add=True)
            return carry

        lax.fori_loop(0, nch, body, 0)
        plsc.subcore_barrier()
        pltpu.sync_copy(deg_sh.at[pl.ds(s * RPT, RPT)],
                        out_hbm.at[pl.ds(c * MDEG + s * RPT, RPT)])

    return hist


def _make_scatter(NPAD, D, E, K, GB):
    EPW = E // NW
    nch = EPW // K
    nblk = nch // GB
    RPT = NPAD // NS
    assert nch % GB == 0 and nblk % 2 == 0 and GB % 2 == 0

    @functools.partial(
        pl.kernel,
        out_type=jax.ShapeDtypeStruct((NC * NPAD, D), jnp.float32),
        mesh=_sc_mesh(),
        scratch_types=[
            pltpu.VMEM_SHARED((NPAD, D), jnp.float32),
            pltpu.VMEM((GB, 2, K), jnp.int32),
            pltpu.VMEM((GB, 2, K), jnp.int32),
            pltpu.VMEM((K, D), jnp.float32),
            pltpu.VMEM((K, D), jnp.float32),
            pltpu.SemaphoreType.DMA,
            pltpu.SemaphoreType.DMA,
            pltpu.SemaphoreType.DMA,
            pltpu.SemaphoreType.DMA,
        ],
    )
    def scat(g_hbm, edg_hbm, zeros_hbm, out_hbm,
             acc_sh, ixa, ixb, rows_a, rows_b, sem_a, sem_b, sem_ia, sem_ib):
        c = lax.axis_index("c")
        s = lax.axis_index("s")
        w = c * NS + s
        pltpu.sync_copy(zeros_hbm.at[pl.ds(s * RPT, RPT)],
                        acc_sh.at[pl.ds(s * RPT, RPT)])
        plsc.subcore_barrier()

        def stage(b, buf, sem):
            return pltpu.async_copy(
                edg_hbm.at[w, pl.ds(b * GB, GB)], buf, sem)

        def wait_stage(buf, sem):
            pltpu.make_async_copy(edg_hbm.at[w, pl.ds(0, GB)], buf, sem).wait()

        def gather(ix, t, buf, sem):
            pltpu.async_copy(g_hbm.at[ix.at[t, 0]], buf, sem)

        def wait_rows(buf, sem):
            pltpu.make_async_copy(g_hbm.at[ixa.at[0, 0]], buf, sem).wait()

        def scatter(ix, t, buf):
            pltpu.sync_copy(buf, acc_sh.at[ix.at[t, 1]], add=True)

        def block(ix, nxt_ix, nxt_sem, has_next):
            def pair(ti, carry):
                t = 2 * ti
                wait_rows(rows_a, sem_a)
                gather(ix, t + 1, rows_b, sem_b)
                scatter(ix, t, rows_a)
                wait_rows(rows_b, sem_b)
                gather(ix, t + 2, rows_a, sem_a)
                scatter(ix, t + 1, rows_b)
                return carry

            lax.fori_loop(0, GB // 2 - 1, pair, 0)
            t = GB - 2
            wait_rows(rows_a, sem_a)
            gather(ix, t + 1, rows_b, sem_b)
            scatter(ix, t, rows_a)
            wait_rows(rows_b, sem_b)

            @pl.when(has_next)
            def _():
                wait_stage(nxt_ix, nxt_sem)
                gather(nxt_ix, 0, rows_a, sem_a)

            scatter(ix, t + 1, rows_b)

        stage(0, ixa, sem_ia).wait()
        stage(1, ixb, sem_ib)
        gather(ixa, 0, rows_a, sem_a)

        def outer2(bi, carry):
            b0 = 2 * bi
            block(ixa, ixb, sem_ib, b0 + 1 < nblk)

            @pl.when(b0 + 2 < nblk)
            def _():
                stage(b0 + 2, ixa, sem_ia)

            block(ixb, ixa, sem_ia, b0 + 2 < nblk)

            @pl.when(b0 + 3 < nblk)
            def _():
                stage(b0 + 3, ixb, sem_ib)

            return carry

        lax.fori_loop(0, nblk // 2, outer2, 0)
        plsc.subcore_barrier()
        pltpu.sync_copy(acc_sh.at[pl.ds(s * RPT, RPT)],
                        out_hbm.at[pl.ds(c * NPAD + s * RPT, RPT)])

    return scat


def _matmul_scale(x, W, d0, d1):
    N, Din = x.shape
    Dout = W.shape[1]
    BN = 1000

    def body(x_ref, w_ref, d0_ref, d1_ref, g_ref, dinv_ref):
        dinv = lax.rsqrt(d0_ref[...] + d1_ref[...] + 1.0)
        h = jnp.dot(x_ref[...], w_ref[...],
                    preferred_element_type=jnp.float32)
        g_ref[...] = h * dinv
        dinv_ref[...] = dinv

    return pl.pallas_call(
        body,
        grid=(N // BN,),
        in_specs=[
            pl.BlockSpec((BN, Din), lambda i: (i, 0)),
            pl.BlockSpec((Din, Dout), lambda i: (0, 0)),
            pl.BlockSpec((BN, 1), lambda i: (i, 0)),
            pl.BlockSpec((BN, 1), lambda i: (i, 0)),
        ],
        out_specs=[
            pl.BlockSpec((BN, Dout), lambda i: (i, 0)),
            pl.BlockSpec((BN, 1), lambda i: (i, 0)),
        ],
        out_shape=[
            jax.ShapeDtypeStruct((N, Dout), jnp.float32),
            jax.ShapeDtypeStruct((N, 1), jnp.float32),
        ],
    )(x, W, d0, d1)


def _final(acc, g, dinv, b2d):
    N = g.shape[0]
    D = g.shape[1]
    BN = 1000

    def body(a_ref, g_ref, dinv_ref, b_ref, o_ref):
        o_ref[...] = (dinv_ref[...] * (a_ref[0] + a_ref[1] + g_ref[...])
                      + b_ref[...])

    return pl.pallas_call(
        body,
        grid=(N // BN,),
        in_specs=[
            pl.BlockSpec((2, BN, D), lambda i: (0, i, 0)),
            pl.BlockSpec((BN, D), lambda i: (i, 0)),
            pl.BlockSpec((BN, 1), lambda i: (i, 0)),
            pl.BlockSpec((1, D), lambda i: (0, 0)),
        ],
        out_specs=pl.BlockSpec((BN, D), lambda i: (i, 0)),
        out_shape=jax.ShapeDtypeStruct((N, D), jnp.float32),
    )(acc, g, dinv, b2d)


def kernel(x, edge_index, t_embed, W, b):
    N, Din = x.shape
    Dout = W.shape[1]
    E = edge_index.shape[1]
    src = edge_index[0]
    dst = edge_index[1]

    NPAD = 10240
    KH = 80
    K = 125
    GB = 8

    nch = E // NW // K
    edg = jnp.stack(
        [src.reshape(NW, nch, K), dst.reshape(NW, nch, K)], axis=2)
    dst3dh = dst.reshape(NW, E // NW // KH, KH)

    degp = _make_hist(E, NPAD, KH)(dst3dh)
    d0 = degp[:N].reshape(N, 1)
    d1 = degp[NPAD:NPAD + N].reshape(N, 1)

    g, dinv = _matmul_scale(x, W, d0, d1)

    zeros2d = jnp.zeros((NPAD, Dout), jnp.float32)
    acc = _make_scatter(NPAD, Dout, E, K, GB)(g, edg, zeros2d)
    acc = acc.reshape(NC, NPAD, Dout)

    out = _final(acc, g, dinv, b.reshape(1, Dout))
    return (out, edge_index, t_embed)
